# baseline (speedup 1.0000x reference)
import functools

import jax
import jax.numpy as jnp
from jax import lax
from jax.experimental import pallas as pl
from jax.experimental.pallas import tpu as pltpu
from jax.experimental.pallas import tpu_sc as plsc

NUM_CORES = 2
NUM_SUBCORES = 16
NUM_WORKERS = NUM_CORES * NUM_SUBCORES
LANES = 16

BATCH = 16384
DIM = 32
B_PER_W = BATCH // NUM_WORKERS
CHUNK = 128
N_CHUNKS = B_PER_W // CHUNK
GROUPS = B_PER_W // LANES


def _mf_body(uidx_hbm, iidx_hbm, uemb_hbm, iemb_hbm, w_hbm, out_hbm,
             idx_u, idx_i, rows_u, rows_i, w_v, out_v, sem):
    wid = lax.axis_index("s") * NUM_CORES + lax.axis_index("c")
    base = wid * B_PER_W

    pltpu.sync_copy(uidx_hbm.at[wid], idx_u)
    pltpu.sync_copy(iidx_hbm.at[wid], idx_i)
    pltpu.sync_copy(w_hbm, w_v)

    cps = []
    for j in range(N_CHUNKS):
        cps.append(pltpu.async_copy(
            uemb_hbm.at[idx_u.at[j]], rows_u.at[pl.ds(j * CHUNK, CHUNK)], sem))
        cps.append(pltpu.async_copy(
            iemb_hbm.at[idx_i.at[j]], rows_i.at[pl.ds(j * CHUNK, CHUNK)], sem))
    for c in cps:
        c.wait()

    iota = lax.iota(jnp.int32, LANES)

    def group_body(g, carry):
        row_ids = jnp.full((LANES,), g * LANES, jnp.int32) + iota
        acc = jnp.zeros((LANES,), jnp.float32)
        for d in range(DIM):
            col_ids = jnp.full((LANES,), d, jnp.int32)
            u16 = plsc.load_gather(rows_u, [row_ids, col_ids])
            i16 = plsc.load_gather(rows_i, [row_ids, col_ids])
            acc = acc + u16 * i16 * w_v[d, :]
        sig = 1.0 / (1.0 + jnp.exp(-acc))
        out_v[pl.ds(g * LANES, LANES)] = sig
        return carry

    lax.fori_loop(0, GROUPS, group_body, 0)

    pltpu.sync_copy(out_v, out_hbm.at[pl.ds(base, B_PER_W)])


@functools.partial(
    pl.kernel,
    out_type=jax.ShapeDtypeStruct((BATCH,), jnp.float32),
    mesh=plsc.VectorSubcoreMesh(core_axis_name="c", subcore_axis_name="s"),
    scratch_types=[
        pltpu.VMEM((N_CHUNKS, CHUNK), jnp.int32),
        pltpu.VMEM((N_CHUNKS, CHUNK), jnp.int32),
        pltpu.VMEM((B_PER_W, DIM), jnp.float32),
        pltpu.VMEM((B_PER_W, DIM), jnp.float32),
        pltpu.VMEM((DIM, LANES), jnp.float32),
        pltpu.VMEM((B_PER_W,), jnp.float32),
        pltpu.SemaphoreType.DMA,
    ],
    compiler_params=pltpu.CompilerParams(
        needs_layout_passes=False, use_tc_tiling_on_sc=False),
)
def _mf_kernel(*refs):
    _mf_body(*refs)


def kernel(user_indices, item_indices, user_emb, item_emb, fc_w):
    uidx = user_indices.astype(jnp.int32).reshape(NUM_WORKERS, N_CHUNKS, CHUNK)
    iidx = item_indices.astype(jnp.int32).reshape(NUM_WORKERS, N_CHUNKS, CHUNK)
    w_b = jnp.broadcast_to(fc_w.reshape(DIM, 1), (DIM, LANES))
    return _mf_kernel(uidx, iidx, user_emb, item_emb, w_b)
